# baseline (speedup 1.0000x reference)
import functools

import jax
import jax.numpy as jnp
from jax import lax
from jax.experimental import pallas as pl
from jax.experimental.pallas import tpu as pltpu
from jax.experimental.pallas import tpu_sc as plsc

_TOKENS = 32768
_HIDDEN = 4096
_EXPERTS = 64
_TOPK = 8

_TC_BLK = 512
_CHUNK_SIZES = (8192, 8192, 8192, 7168, 1024)

_NCORES = 2
_NSUB = 16
_NW = _NCORES * _NSUB


def _router_scores_body(x_ref, w_ref, out_ref):
    logits = lax.dot_general(
        x_ref[...], w_ref[...],
        dimension_numbers=(((1,), (1,)), ((), ())),
        preferred_element_type=jnp.float32,
    )
    m = jnp.max(logits, axis=-1, keepdims=True)
    e = jnp.exp(logits - m)
    scores = e / jnp.sum(e, axis=-1, keepdims=True)
    out_ref[...] = scores


def _router_scores(x, W, base_tokens, chunk_tokens):
    base = base_tokens // _TC_BLK
    grid = (chunk_tokens // _TC_BLK,)
    return pl.pallas_call(
        _router_scores_body,
        grid=grid,
        in_specs=[
            pl.BlockSpec((_TC_BLK, _HIDDEN), lambda i: (base + i, 0)),
            pl.BlockSpec((_EXPERTS, _HIDDEN), lambda i: (0, 0)),
        ],
        out_specs=pl.BlockSpec((_TC_BLK, _EXPERTS), lambda i: (i, 0)),
        out_shape=jax.ShapeDtypeStruct((chunk_tokens, _EXPERTS), jnp.float32),
        compiler_params=pltpu.CompilerParams(
            dimension_semantics=("parallel",),
        ),
    )(x, W)


def _topk_sc(scores_flat, chunk_tokens):
    rows_w = chunk_tokens // _NW
    scores_w = rows_w * _EXPERTS
    out_w = rows_w * _TOPK
    mesh = plsc.VectorSubcoreMesh(core_axis_name="c", subcore_axis_name="s")

    @functools.partial(
        pl.kernel,
        mesh=mesh,
        out_type=[
            jax.ShapeDtypeStruct((chunk_tokens * _TOPK,), jnp.float32),
            jax.ShapeDtypeStruct((chunk_tokens * _TOPK,), jnp.int32),
        ],
        scratch_types=[
            pltpu.VMEM((scores_w,), jnp.float32),
            pltpu.VMEM((out_w + 16,), jnp.float32),
            pltpu.VMEM((out_w + 16,), jnp.int32),
        ],
        compiler_params=pltpu.CompilerParams(needs_layout_passes=False),
    )
    def topk_kernel(scores_hbm, w_hbm, i_hbm, s_v, w_v, i_v):
        wid = lax.axis_index("s") * _NCORES + lax.axis_index("c")
        pltpu.sync_copy(scores_hbm.at[pl.ds(wid * scores_w, scores_w)], s_v)

        lane = lax.iota(jnp.int32, 16)
        m8 = lane < 8
        zero16 = jnp.zeros((16,), jnp.float32)

        def merge(kA, vA, kB, vB):
            k = jnp.where(m8, kA, lax.rev(kB, (0,)))
            v = jnp.where(m8, vA, lax.rev(vB, (0,)))
            return plsc.sort_key_val(k, v, descending=True)

        def row_body(r, carry):
            ks, vs = [], []
            for c in range(4):
                s = s_v[pl.ds(r * _EXPERTS + c * 16, 16)]
                sk, sv = plsc.sort_key_val(s, lane + c * 16, descending=True)
                ks.append(sk)
                vs.append(sv)
            ka, va = merge(ks[0], vs[0], ks[1], vs[1])
            kb, vb = merge(ks[2], vs[2], ks[3], vs[3])
            kc, vc = merge(ka, va, kb, vb)
            ssum = jnp.sum(jnp.where(m8, kc, zero16))
            wf = kc / ssum
            plsc.store_compressed(w_v.at[pl.ds(r * _TOPK, 16)], wf, mask=m8)
            plsc.store_compressed(i_v.at[pl.ds(r * _TOPK, 16)], vc, mask=m8)
            return carry

        lax.fori_loop(0, rows_w, row_body, 0)

        pltpu.sync_copy(w_v.at[pl.ds(0, out_w)],
                        w_hbm.at[pl.ds(wid * out_w, out_w)])
        pltpu.sync_copy(i_v.at[pl.ds(0, out_w)],
                        i_hbm.at[pl.ds(wid * out_w, out_w)])

    return topk_kernel(scores_flat)


def kernel(x, W):
    x2 = x.reshape(-1, x.shape[-1])
    scores_parts, w_parts, i_parts = [], [], []
    base = 0
    for chunk_tokens in _CHUNK_SIZES:
        sc = _router_scores(x2, W, base, chunk_tokens)
        scores_parts.append(sc)
        w_flat, i_flat = _topk_sc(sc.reshape(-1), chunk_tokens)
        w_parts.append(w_flat.reshape(chunk_tokens, _TOPK))
        i_parts.append(i_flat.reshape(chunk_tokens, _TOPK))
        base += chunk_tokens
    scores = jnp.concatenate(scores_parts, axis=0)
    expert_weights = jnp.concatenate(w_parts, axis=0)
    expert_indices = jnp.concatenate(i_parts, axis=0)
    return (scores, expert_weights, expert_indices)


# speedup vs baseline: 1.0196x; 1.0196x over previous
import functools

import jax
import jax.numpy as jnp
from jax import lax
from jax.experimental import pallas as pl
from jax.experimental.pallas import tpu as pltpu
from jax.experimental.pallas import tpu_sc as plsc

_TOKENS = 32768
_HIDDEN = 4096
_EXPERTS = 64
_TOPK = 8

_TC_BLK = 512
_CHUNK_SIZES = (10240, 10240, 10240, 2048)

_NCORES = 2
_NSUB = 16
_NW = _NCORES * _NSUB


def _router_scores_body(x_ref, w_ref, out_ref):
    logits = lax.dot_general(
        x_ref[...], w_ref[...],
        dimension_numbers=(((1,), (1,)), ((), ())),
        preferred_element_type=jnp.float32,
    )
    m = jnp.max(logits, axis=-1, keepdims=True)
    e = jnp.exp(logits - m)
    scores = e / jnp.sum(e, axis=-1, keepdims=True)
    out_ref[...] = scores


def _router_scores(x, W, base_tokens, chunk_tokens):
    base = base_tokens // _TC_BLK
    grid = (chunk_tokens // _TC_BLK,)
    return pl.pallas_call(
        _router_scores_body,
        grid=grid,
        in_specs=[
            pl.BlockSpec((_TC_BLK, _HIDDEN), lambda i: (base + i, 0)),
            pl.BlockSpec((_EXPERTS, _HIDDEN), lambda i: (0, 0)),
        ],
        out_specs=pl.BlockSpec((_TC_BLK, _EXPERTS), lambda i: (i, 0)),
        out_shape=jax.ShapeDtypeStruct((chunk_tokens, _EXPERTS), jnp.float32),
        compiler_params=pltpu.CompilerParams(
            dimension_semantics=("parallel",),
        ),
    )(x, W)


def _topk_sc(scores_flat, chunk_tokens):
    rows_w = chunk_tokens // _NW
    scores_w = rows_w * _EXPERTS
    out_w = rows_w * _TOPK
    mesh = plsc.VectorSubcoreMesh(core_axis_name="c", subcore_axis_name="s")

    @functools.partial(
        pl.kernel,
        mesh=mesh,
        out_type=[
            jax.ShapeDtypeStruct((chunk_tokens * _TOPK,), jnp.float32),
            jax.ShapeDtypeStruct((chunk_tokens * _TOPK,), jnp.int32),
        ],
        scratch_types=[
            pltpu.VMEM((scores_w,), jnp.float32),
            pltpu.VMEM((out_w + 16,), jnp.float32),
            pltpu.VMEM((out_w + 16,), jnp.int32),
        ],
        compiler_params=pltpu.CompilerParams(needs_layout_passes=False),
    )
    def topk_kernel(scores_hbm, w_hbm, i_hbm, s_v, w_v, i_v):
        wid = lax.axis_index("s") * _NCORES + lax.axis_index("c")
        pltpu.sync_copy(scores_hbm.at[pl.ds(wid * scores_w, scores_w)], s_v)

        lane = lax.iota(jnp.int32, 16)
        m8 = lane < 8
        zero16 = jnp.zeros((16,), jnp.float32)

        def merge(kA, vA, kB, vB):
            k = jnp.where(m8, kA, lax.rev(kB, (0,)))
            v = jnp.where(m8, vA, lax.rev(vB, (0,)))
            return plsc.sort_key_val(k, v, descending=True)

        def row_body(r, carry):
            ks, vs = [], []
            for c in range(4):
                s = s_v[pl.ds(r * _EXPERTS + c * 16, 16)]
                sk, sv = plsc.sort_key_val(s, lane + c * 16, descending=True)
                ks.append(sk)
                vs.append(sv)
            ka, va = merge(ks[0], vs[0], ks[1], vs[1])
            kb, vb = merge(ks[2], vs[2], ks[3], vs[3])
            kc, vc = merge(ka, va, kb, vb)
            ssum = jnp.sum(jnp.where(m8, kc, zero16))
            wf = kc / ssum
            plsc.store_compressed(w_v.at[pl.ds(r * _TOPK, 16)], wf, mask=m8)
            plsc.store_compressed(i_v.at[pl.ds(r * _TOPK, 16)], vc, mask=m8)
            return carry

        lax.fori_loop(0, rows_w, row_body, 0)

        pltpu.sync_copy(w_v.at[pl.ds(0, out_w)],
                        w_hbm.at[pl.ds(wid * out_w, out_w)])
        pltpu.sync_copy(i_v.at[pl.ds(0, out_w)],
                        i_hbm.at[pl.ds(wid * out_w, out_w)])

    return topk_kernel(scores_flat)


def kernel(x, W):
    x2 = x.reshape(-1, x.shape[-1])
    scores_parts, w_parts, i_parts = [], [], []
    base = 0
    for chunk_tokens in _CHUNK_SIZES:
        sc = _router_scores(x2, W, base, chunk_tokens)
        scores_parts.append(sc)
        w_flat, i_flat = _topk_sc(sc.reshape(-1), chunk_tokens)
        w_parts.append(w_flat.reshape(chunk_tokens, _TOPK))
        i_parts.append(i_flat.reshape(chunk_tokens, _TOPK))
        base += chunk_tokens
    scores = jnp.concatenate(scores_parts, axis=0)
    expert_weights = jnp.concatenate(w_parts, axis=0)
    expert_indices = jnp.concatenate(i_parts, axis=0)
    return (scores, expert_weights, expert_indices)
